# Initial kernel scaffold; baseline (speedup 1.0000x reference)
import jax
import jax.numpy as jnp
from jax.experimental import pallas as pl


def kernel(x, pos, batch, kernel, kernel_weight):
    raise NotImplementedError("write your pallas kernel here")


# speedup vs baseline: 1.4059x; 1.4059x over previous
import functools
import math

import jax
import jax.numpy as jnp
from jax.experimental import pallas as pl
from jax.experimental.pallas import tpu as pltpu

N = 10000
IN_F = 128
OUT_F = 128
P = 16
RADIUS = 0.2
KP_EXTENT = RADIUS / 1.5
Q = int(math.ceil(0.1 * N))
QPAD = 1024
EPAD = P * QPAD


def _fps_xla(pos):
    sampled0 = jnp.zeros((Q,), dtype=jnp.int32)
    mind0 = jnp.sum((pos - pos[0]) ** 2, axis=1)

    def body(i, state):
        sampled, mind = state
        nxt = jnp.argmax(mind).astype(jnp.int32)
        sampled = sampled.at[i].set(nxt)
        d = jnp.sum((pos - pos[nxt]) ** 2, axis=1)
        mind = jnp.minimum(mind, d)
        return (sampled, mind)

    sampled, _ = jax.lax.fori_loop(1, Q, body, (sampled0, mind0))
    return sampled


def _neighbors_xla(pos, idx):
    q = pos[idx]
    d2 = (jnp.sum(q * q, axis=1)[:, None]
          + jnp.sum(pos * pos, axis=1)[None, :]
          - 2.0 * (q @ pos.T))
    d2 = jnp.where(d2 <= RADIUS * RADIUS, d2, jnp.inf)
    _, cols = jax.lax.top_k(-d2, P)
    return cols


def _message_body(pos_i_ref, posg_ref, kT_ref, xg_ref, w_ref, out_ref):
    kiota = jax.lax.broadcasted_iota(jnp.int32, (QPAD, P), 1)
    c = jnp.zeros((QPAD, P), dtype=jnp.float32)
    for p in range(P):
        pg = posg_ref[pl.ds(p * QPAD, QPAD), :]
        nx = pos_i_ref[:, 0:1] - pg[:, 0:1]
        ny = pos_i_ref[:, 1:2] - pg[:, 1:2]
        nz = pos_i_ref[:, 2:3] - pg[:, 2:3]
        dx = nx - kT_ref[0:1, :]
        dy = ny - kT_ref[1:2, :]
        dz = nz - kT_ref[2:3, :]
        sq = dx * dx + dy * dy + dz * dz
        m = jnp.min(sq, axis=1, keepdims=True)
        kstar = jnp.min(jnp.where(sq == m, kiota, P + 1), axis=1, keepdims=True)
        w = 1.0 - jnp.sqrt(m) / KP_EXTENT
        w = jnp.where(w < 0.0, 0.0, w)
        c = c + jnp.where(kstar == kiota, w, 0.0)
    acc = jnp.zeros((QPAD, OUT_F), dtype=jnp.float32)
    for k in range(P):
        xk = xg_ref[pl.ds(k * QPAD, QPAD), :]
        zk = c[:, k:k + 1] * xk
        acc = acc + jax.lax.dot_general(
            zk, w_ref[k],
            (((1,), (0,)), ((), ())),
            preferred_element_type=jnp.float32,
            precision=jax.lax.Precision.HIGHEST)
    out_ref[...] = acc


@jax.jit
def _message_pallas(pos_i_pad, posg, kT, xg, kernel_weight):
    return pl.pallas_call(
        _message_body,
        out_shape=jax.ShapeDtypeStruct((QPAD, OUT_F), jnp.float32),
    )(pos_i_pad, posg, kT, xg, kernel_weight)


def kernel(x, pos, batch, kernel, kernel_weight):
    idx = _fps_xla(pos)
    cols = _neighbors_xla(pos, idx)
    colkm = jnp.zeros((P, QPAD), dtype=jnp.int32)
    colkm = colkm.at[:, :Q].set(cols.T)
    colkm = colkm.reshape(-1)
    xg = jnp.take(x, colkm, axis=0)
    posg = jnp.zeros((EPAD, 16), jnp.float32).at[:, :3].set(jnp.take(pos, colkm, axis=0))
    pos_i_pad = jnp.zeros((QPAD, 16), jnp.float32).at[:Q, :3].set(pos[:Q])
    kT = jnp.zeros((8, P), jnp.float32).at[:3, :].set(kernel.reshape(P, 3).T)
    dout = _message_pallas(pos_i_pad, posg, kT, xg, kernel_weight)
    out = jnp.zeros((N, OUT_F), jnp.float32).at[:Q, :].set(dout[:Q])
    return (out, pos[idx], batch[idx])


# speedup vs baseline: 8.3618x; 5.9476x over previous
import functools
import math

import jax
import jax.numpy as jnp
from jax.experimental import pallas as pl
from jax.experimental.pallas import tpu as pltpu

N = 10000
IN_F = 128
OUT_F = 128
P = 16
RADIUS = 0.2
KP_EXTENT = RADIUS / 1.5
Q = int(math.ceil(0.1 * N))
QPAD = 1024
EPAD = P * QPAD


NPAD = 10240


def _fps_body(posT_ref, rows_ref, idx_ref, posq_ref):
    ridx = jax.lax.broadcasted_iota(jnp.int32, (8, 1280), 0)
    lidx = jax.lax.broadcasted_iota(jnp.int32, (8, 1280), 1)
    gidx = ridx * 1280 + lidx
    px = posT_ref[0]
    py = posT_ref[1]
    pz = posT_ref[2]

    def dist(j):
        row = rows_ref[pl.ds(j, 1), :]
        dx = px - row[:, 0:1]
        dy = py - row[:, 1:2]
        dz = pz - row[:, 2:3]
        return dx * dx + dy * dy + dz * dz, row

    idx_ref[0] = 0
    d0, row0 = dist(0)
    posq_ref[pl.ds(0, 1), :] = row0
    mind0 = jnp.where(gidx < N, d0, -jnp.inf)

    def body(i, mind):
        m = jnp.max(mind)
        j = jnp.min(jnp.where(mind == m, gidx, jnp.int32(2**30)))
        idx_ref[i] = j
        d, row = dist(j)
        posq_ref[pl.ds(i, 1), :] = row
        return jnp.minimum(mind, d)

    jax.lax.fori_loop(1, Q, body, mind0, unroll=False)


@jax.jit
def _fps_pallas(posT, rows):
    return pl.pallas_call(
        _fps_body,
        out_shape=(jax.ShapeDtypeStruct((Q,), jnp.int32),
                   jax.ShapeDtypeStruct((Q, 3), jnp.float32)),
        out_specs=(pl.BlockSpec(memory_space=pltpu.SMEM),
                   pl.BlockSpec()),
    )(posT, rows)


def _fps_xla(pos):
    sampled0 = jnp.zeros((Q,), dtype=jnp.int32)
    mind0 = jnp.sum((pos - pos[0]) ** 2, axis=1)

    def body(i, state):
        sampled, mind = state
        nxt = jnp.argmax(mind).astype(jnp.int32)
        sampled = sampled.at[i].set(nxt)
        d = jnp.sum((pos - pos[nxt]) ** 2, axis=1)
        mind = jnp.minimum(mind, d)
        return (sampled, mind)

    sampled, _ = jax.lax.fori_loop(1, Q, body, (sampled0, mind0))
    return sampled


def _d2_xla(pos, q):
    d2 = (jnp.sum(q * q, axis=1)[:, None]
          + jnp.sum(pos * pos, axis=1)[None, :]
          - 2.0 * (q @ pos.T))
    return jnp.where(d2 <= RADIUS * RADIUS, d2, jnp.inf)


def _select_body(d2_ref, cols_ref):
    neg = -d2_ref[...]
    li = jax.lax.broadcasted_iota(jnp.int32, (8, N), 1)
    BIG = jnp.int32(2**30)
    for t in range(P):
        m = jnp.max(neg, axis=1, keepdims=True)
        j = jnp.min(jnp.where(neg == m, li, BIG), axis=1, keepdims=True)
        cols_ref[:, t:t + 1] = j
        hit = li == j
        neg = jnp.where(hit, -jnp.inf, neg)
        li = jnp.where(hit, BIG, li)


@jax.jit
def _select_pallas(d2m):
    return pl.pallas_call(
        _select_body,
        grid=(Q // 8,),
        in_specs=[pl.BlockSpec((8, N), lambda i: (i, 0))],
        out_specs=pl.BlockSpec((8, P), lambda i: (i, 0)),
        out_shape=jax.ShapeDtypeStruct((Q, P), jnp.int32),
    )(d2m)


def _message_body(pos_i_ref, posg_ref, kT_ref, xg_ref, w_ref, out_ref):
    kiota = jax.lax.broadcasted_iota(jnp.int32, (QPAD, P), 1)
    c = jnp.zeros((QPAD, P), dtype=jnp.float32)
    for p in range(P):
        pg = posg_ref[pl.ds(p * QPAD, QPAD), :]
        nx = pos_i_ref[:, 0:1] - pg[:, 0:1]
        ny = pos_i_ref[:, 1:2] - pg[:, 1:2]
        nz = pos_i_ref[:, 2:3] - pg[:, 2:3]
        dx = nx - kT_ref[0:1, :]
        dy = ny - kT_ref[1:2, :]
        dz = nz - kT_ref[2:3, :]
        sq = dx * dx + dy * dy + dz * dz
        m = jnp.min(sq, axis=1, keepdims=True)
        kstar = jnp.min(jnp.where(sq == m, kiota, P + 1), axis=1, keepdims=True)
        w = 1.0 - jnp.sqrt(m) / KP_EXTENT
        w = jnp.where(w < 0.0, 0.0, w)
        c = c + jnp.where(kstar == kiota, w, 0.0)
    acc = jnp.zeros((QPAD, OUT_F), dtype=jnp.float32)
    for k in range(P):
        xk = xg_ref[pl.ds(k * QPAD, QPAD), :]
        zk = c[:, k:k + 1] * xk
        acc = acc + jax.lax.dot_general(
            zk, w_ref[k],
            (((1,), (0,)), ((), ())),
            preferred_element_type=jnp.float32,
            precision=jax.lax.Precision.HIGHEST)
    out_ref[...] = acc


@jax.jit
def _message_pallas(pos_i_pad, posg, kT, xg, kernel_weight):
    return pl.pallas_call(
        _message_body,
        out_shape=jax.ShapeDtypeStruct((QPAD, OUT_F), jnp.float32),
    )(pos_i_pad, posg, kT, xg, kernel_weight)


def kernel(x, pos, batch, kernel, kernel_weight):
    rows = jnp.full((NPAD, 3), 1e6, jnp.float32).at[:N].set(pos)
    posT = rows.T.reshape(3, 8, 1280)
    idx, posq = _fps_pallas(posT, rows)
    cols = _select_pallas(_d2_xla(pos, posq))
    colkm = jnp.zeros((P, QPAD), dtype=jnp.int32)
    colkm = colkm.at[:, :Q].set(cols.T)
    colkm = colkm.reshape(-1)
    xg = jnp.take(x, colkm, axis=0)
    posg = jnp.zeros((EPAD, 16), jnp.float32).at[:, :3].set(jnp.take(pos, colkm, axis=0))
    pos_i_pad = jnp.zeros((QPAD, 16), jnp.float32).at[:Q, :3].set(pos[:Q])
    kT = jnp.zeros((8, P), jnp.float32).at[:3, :].set(kernel.reshape(P, 3).T)
    dout = _message_pallas(pos_i_pad, posg, kT, xg, kernel_weight)
    out = jnp.zeros((N, OUT_F), jnp.float32).at[:Q, :].set(dout[:Q])
    return (out, posq, batch[idx])
